# baseline (speedup 1.0000x reference)
import functools

import jax
import jax.numpy as jnp
from jax import lax
from jax.experimental import pallas as pl
from jax.experimental.pallas import tpu as pltpu
from jax.experimental.pallas import tpu_sc as plsc

N = 16384
P = 20
V = 1000000
DV = 1000
NC, NS = 2, 16
NH = N // NC
NSW = NH // NS
NB = NSW // 128
CW = 128
NFC = V // CW
REM = V - NFC * CW
NCH = NFC + 1
HMAX = NH + 16
RR = N + 64


def _sc_embed(road_t, dt_t, xr, xd):
  mesh = plsc.VectorSubcoreMesh(core_axis_name="c", subcore_axis_name="s")

  @functools.partial(
      pl.kernel,
      mesh=mesh,
      compiler_params=pltpu.CompilerParams(use_tc_tiling_on_sc=True,
                                           needs_layout_passes=False),
      out_type=(jax.ShapeDtypeStruct((P, 64, N), jnp.float32),
                jax.ShapeDtypeStruct((RR, 128), jnp.float32)),
      scratch_types=[
          pltpu.VMEM((NH,), jnp.int32),
          pltpu.VMEM((HMAX,), jnp.int32),
          pltpu.VMEM((2, 32, CW), jnp.float32),
          pltpu.VMEM((32, REM), jnp.float32),
          pltpu.VMEM((16, 128), jnp.float32),
          pltpu.VMEM((16,), jnp.int32),
          pltpu.VMEM((32, DV), jnp.float32),
          pltpu.VMEM((NSW,), jnp.int32),
          pltpu.VMEM((128, 128), jnp.float32),
          pltpu.VMEM((4, 8, NB * 128), jnp.float32),
          pltpu.VMEM((2, 8, NB * 128), jnp.float32),
          pltpu.SemaphoreType.DMA((2,)),
          pltpu.SemaphoreType.DMA,
          pltpu.SemaphoreType.DMA((2,)),
      ],
  )
  def k(road_hbm, dt_hbm, xr_hbm, xd_hbm, out_hbm, rows_hbm,
        xr_v, hn, cbuf, pbuf, ext, tgt_v, dtv, xd_v, rbuf, roadblk, dtblk,
        ssem, wsem, dsem):
    cid = lax.axis_index("c")
    sid = lax.axis_index("s")
    iota = lax.iota(jnp.int32, 16)

    pltpu.sync_copy(xr_hbm.at[pl.ds(cid * NH, NH)], xr_v)
    pltpu.sync_copy(dt_hbm, dtv)

    c0 = sid * NCH // NS
    c1 = (sid + 1) * NCH // NS
    lane_lo = c0 * CW
    lane_hi = c1 * CW

    def scan_body(v, cnt):
      idx = xr_v[pl.ds(v * 16, 16)]
      m = (idx >= lane_lo) & (idx < lane_hi)
      pos = cnt + plsc.cumsum(m.astype(jnp.int32)) - 1
      plsc.store_scatter(hn, [pos], iota + v * 16, mask=m)
      return cnt + jnp.sum(m.astype(jnp.int32))

    cnt = lax.fori_loop(0, NH // 16, scan_body, jnp.int32(0))
    hn[pl.ds(cnt, 16)] = jnp.zeros((16,), jnp.int32)
    nhv = (cnt + 15) // 16

    def process_chunk(bufref, k_lane0):
      def hit_body(hv, carry):
        nl = hn[pl.ds(hv * 16, 16)]
        hi = plsc.load_gather(xr_v, [nl])
        m = (hi >= k_lane0) & (hi < k_lane0 + CW)

        def extract():
          local = jnp.where(m, hi - k_lane0, 0)

          def c_body(c, carry2):
            vals = plsc.load_gather(bufref, [jnp.full((16,), c, jnp.int32),
                                             local])
            plsc.store_scatter(ext, [iota, jnp.full((16,), c, jnp.int32)],
                               vals)
            return carry2

          lax.fori_loop(0, 32, c_body, 0)
          tgt_v[...] = jnp.where(m, cid * NH + nl, N + iota)
          pltpu.sync_copy(ext, rows_hbm.at[tgt_v])

        pl.when(jnp.sum(m.astype(jnp.int32)) > 0)(extract)
        return carry

      lax.fori_loop(0, nhv, hit_body, 0)

    nch_full = jnp.minimum(c1, NFC) - c0

    def issue(kc, par):
      pltpu.async_copy(road_hbm.at[:, pl.ds(kc * CW, CW)], cbuf.at[par],
                       ssem.at[par])

    def wait(par):
      pltpu.make_async_copy(road_hbm.at[:, pl.ds(0, CW)], cbuf.at[par],
                            ssem.at[par]).wait()

    pl.when(nch_full > 0)(lambda: issue(c0, 0))

    def sweep_body(i, carry):
      par = lax.rem(i, 2)
      pl.when(i + 1 < nch_full)(lambda: issue(c0 + i + 1, 1 - par))
      wait(par)
      process_chunk(cbuf.at[par], (c0 + i) * CW)
      return carry

    lax.fori_loop(0, nch_full, sweep_body, 0)

    def partial_chunk():
      pltpu.sync_copy(road_hbm.at[:, pl.ds(NFC * CW, REM)], pbuf)
      process_chunk(pbuf, NFC * CW)

    pl.when(c1 == NCH)(partial_chunk)

    n0 = cid * NH + sid * NSW

    def p_body(p, carry):
      pltpu.sync_copy(xd_hbm.at[pl.ds(p * N + n0, NSW)], xd_v)

      def cb_body(cb, carry2):
        q = lax.rem(p * 4 + cb, 2)
        i2 = p * 4 + cb
        pl.when(i2 >= 2)(lambda: pltpu.make_async_copy(
            dtblk.at[q], out_hbm.at[0, pl.ds(32, 8), pl.ds(n0, NB * 128)],
            dsem.at[q]).wait())

        def r_body(r, carry3):
          def g_body(g, carry4):
            idxv = xd_v[pl.ds(g * 16, 16)]
            vals = plsc.load_gather(dtv, [jnp.full((16,), cb * 8 + r,
                                                   jnp.int32), idxv])
            dtblk[q, r, pl.ds(g * 16, 16)] = vals
            return carry4

          lax.fori_loop(0, NB * 8, g_body, 0)
          return carry3

        lax.fori_loop(0, 8, r_body, 0)
        pltpu.async_copy(dtblk.at[q],
                         out_hbm.at[p, pl.ds(32 + cb * 8, 8),
                                    pl.ds(n0, NB * 128)],
                         dsem.at[q])
        return carry2

      lax.fori_loop(0, 4, cb_body, 0)
      return carry

    lax.fori_loop(0, P, p_body, 0)
    for q in range(2):
      pltpu.make_async_copy(
          dtblk.at[q], out_hbm.at[0, pl.ds(32, 8), pl.ds(n0, NB * 128)],
          dsem.at[q]).wait()

    plsc.subcore_barrier()

    def nb_body(nb, carry):
      pltpu.sync_copy(rows_hbm.at[pl.ds(n0 + nb * 128, 128)], rbuf)

      def cr_body(cr, carry2):
        cb, r = cr // 8, lax.rem(cr, 8)

        def g_body(g, carry3):
          lanes = g * 16 + iota
          vals = plsc.load_gather(rbuf, [lanes,
                                         jnp.full((16,), cr, jnp.int32)])
          roadblk[cb, r, pl.ds(nb * 128 + g * 16, 16)] = vals
          return carry3

        lax.fori_loop(0, 8, g_body, 0)
        return carry2

      lax.fori_loop(0, 32, cr_body, 0)
      return carry

    lax.fori_loop(0, NB, nb_body, 0)

    def pw_body(p, carry):
      for cb in range(4):
        pltpu.async_copy(roadblk.at[cb],
                         out_hbm.at[p, pl.ds(cb * 8, 8), pl.ds(n0, NB * 128)],
                         wsem)
      for cb in range(4):
        pltpu.make_async_copy(
            roadblk.at[cb],
            out_hbm.at[p, pl.ds(cb * 8, 8), pl.ds(n0, NB * 128)],
            wsem).wait()
      return carry

    lax.fori_loop(0, P, pw_body, 0)

  return k(road_t, dt_t, xr, xd)


def kernel(x_road, x_datetime, road_table, datetime_table):
  road_t = road_table.T
  dt_t = datetime_table.T
  xr = x_road.reshape(N).astype(jnp.int32)
  xd = jnp.swapaxes(x_datetime, 0, 1).reshape(P * N).astype(jnp.int32)
  out, _ = _sc_embed(road_t, dt_t, xr, xd)
  return jnp.transpose(out, (2, 0, 1))


# speedup vs baseline: 1.0438x; 1.0438x over previous
import functools

import jax
import jax.numpy as jnp
from jax import lax
from jax.experimental import pallas as pl
from jax.experimental.pallas import tpu as pltpu
from jax.experimental.pallas import tpu_sc as plsc

N = 16384
P = 20
V = 1000000
DV = 1000
NC, NS = 2, 16
NH = N // NC
NSW = NH // NS
NB = NSW // 128
CW = 256
NFC = V // CW
REM = V - NFC * CW
NCH = NFC + 1
HMAX = NH + 16
RR = N + 64


def _sc_embed(road_t, dt_t, xr, xd):
  mesh = plsc.VectorSubcoreMesh(core_axis_name="c", subcore_axis_name="s")

  @functools.partial(
      pl.kernel,
      mesh=mesh,
      compiler_params=pltpu.CompilerParams(use_tc_tiling_on_sc=True,
                                           needs_layout_passes=False),
      out_type=(jax.ShapeDtypeStruct((P, 64, N), jnp.float32),
                jax.ShapeDtypeStruct((RR, 128), jnp.float32)),
      scratch_types=[
          pltpu.VMEM((NH,), jnp.int32),
          pltpu.VMEM((HMAX,), jnp.int32),
          pltpu.VMEM((2, 32, CW), jnp.float32),
          pltpu.VMEM((32, REM), jnp.float32),
          pltpu.VMEM((16, 128), jnp.float32),
          pltpu.VMEM((16,), jnp.int32),
          pltpu.VMEM((32, DV), jnp.float32),
          pltpu.VMEM((NSW,), jnp.int32),
          pltpu.VMEM((128, 128), jnp.float32),
          pltpu.VMEM((4, 8, NB * 128), jnp.float32),
          pltpu.VMEM((2, 8, NB * 128), jnp.float32),
          pltpu.SemaphoreType.DMA((2,)),
          pltpu.SemaphoreType.DMA,
          pltpu.SemaphoreType.DMA((2,)),
      ],
  )
  def k(road_hbm, dt_hbm, xr_hbm, xd_hbm, out_hbm, rows_hbm,
        xr_v, hn, cbuf, pbuf, ext, tgt_v, dtv, xd_v, rbuf, roadblk, dtblk,
        ssem, wsem, dsem):
    cid = lax.axis_index("c")
    sid = lax.axis_index("s")
    iota = lax.iota(jnp.int32, 16)

    pltpu.sync_copy(xr_hbm.at[pl.ds(cid * NH, NH)], xr_v)
    pltpu.sync_copy(dt_hbm, dtv)

    c0 = sid * NCH // NS
    c1 = (sid + 1) * NCH // NS
    lane_lo = c0 * CW
    lane_hi = c1 * CW

    def scan_body(v, cnt):
      idx = xr_v[pl.ds(v * 16, 16)]
      m = (idx >= lane_lo) & (idx < lane_hi)
      pos = cnt + plsc.cumsum(m.astype(jnp.int32)) - 1
      plsc.store_scatter(hn, [pos], iota + v * 16, mask=m)
      return cnt + jnp.sum(m.astype(jnp.int32))

    cnt = lax.fori_loop(0, NH // 16, scan_body, jnp.int32(0))
    hn[pl.ds(cnt, 16)] = jnp.zeros((16,), jnp.int32)
    nhv = (cnt + 15) // 16

    def process_chunk(bufref, k_lane0):
      def hit_body(hv, carry):
        nl = hn[pl.ds(hv * 16, 16)]
        hi = plsc.load_gather(xr_v, [nl])
        m = (hi >= k_lane0) & (hi < k_lane0 + CW)

        def extract():
          local = jnp.where(m, hi - k_lane0, 0)

          def c_body(c, carry2):
            vals = plsc.load_gather(bufref, [jnp.full((16,), c, jnp.int32),
                                             local])
            plsc.store_scatter(ext, [iota, jnp.full((16,), c, jnp.int32)],
                               vals)
            return carry2

          lax.fori_loop(0, 32, c_body, 0)
          tgt_v[...] = jnp.where(m, cid * NH + nl, N + iota)
          pltpu.sync_copy(ext, rows_hbm.at[tgt_v])

        pl.when(jnp.sum(m.astype(jnp.int32)) > 0)(extract)
        return carry

      lax.fori_loop(0, nhv, hit_body, 0)

    nch_full = jnp.minimum(c1, NFC) - c0

    def issue(kc, par):
      pltpu.async_copy(road_hbm.at[:, pl.ds(kc * CW, CW)], cbuf.at[par],
                       ssem.at[par])

    def wait(par):
      pltpu.make_async_copy(road_hbm.at[:, pl.ds(0, CW)], cbuf.at[par],
                            ssem.at[par]).wait()

    pl.when(nch_full > 0)(lambda: issue(c0, 0))

    def sweep_body(i, carry):
      par = lax.rem(i, 2)
      pl.when(i + 1 < nch_full)(lambda: issue(c0 + i + 1, 1 - par))
      wait(par)
      process_chunk(cbuf.at[par], (c0 + i) * CW)
      return carry

    lax.fori_loop(0, nch_full, sweep_body, 0)

    def partial_chunk():
      pltpu.sync_copy(road_hbm.at[:, pl.ds(NFC * CW, REM)], pbuf)
      process_chunk(pbuf, NFC * CW)

    pl.when(c1 == NCH)(partial_chunk)

    n0 = cid * NH + sid * NSW

    def p_body(p, carry):
      pltpu.sync_copy(xd_hbm.at[pl.ds(p * N + n0, NSW)], xd_v)

      def cb_body(cb, carry2):
        q = lax.rem(p * 4 + cb, 2)
        i2 = p * 4 + cb
        pl.when(i2 >= 2)(lambda: pltpu.make_async_copy(
            dtblk.at[q], out_hbm.at[0, pl.ds(32, 8), pl.ds(n0, NB * 128)],
            dsem.at[q]).wait())

        def r_body(r, carry3):
          def g_body(g, carry4):
            idxv = xd_v[pl.ds(g * 16, 16)]
            vals = plsc.load_gather(dtv, [jnp.full((16,), cb * 8 + r,
                                                   jnp.int32), idxv])
            dtblk[q, r, pl.ds(g * 16, 16)] = vals
            return carry4

          lax.fori_loop(0, NB * 8, g_body, 0)
          return carry3

        lax.fori_loop(0, 8, r_body, 0)
        pltpu.async_copy(dtblk.at[q],
                         out_hbm.at[p, pl.ds(32 + cb * 8, 8),
                                    pl.ds(n0, NB * 128)],
                         dsem.at[q])
        return carry2

      lax.fori_loop(0, 4, cb_body, 0)
      return carry

    lax.fori_loop(0, P, p_body, 0)
    for q in range(2):
      pltpu.make_async_copy(
          dtblk.at[q], out_hbm.at[0, pl.ds(32, 8), pl.ds(n0, NB * 128)],
          dsem.at[q]).wait()

    plsc.subcore_barrier()

    def nb_body(nb, carry):
      pltpu.sync_copy(rows_hbm.at[pl.ds(n0 + nb * 128, 128)], rbuf)

      def cr_body(cr, carry2):
        cb, r = cr // 8, lax.rem(cr, 8)

        def g_body(g, carry3):
          lanes = g * 16 + iota
          vals = plsc.load_gather(rbuf, [lanes,
                                         jnp.full((16,), cr, jnp.int32)])
          roadblk[cb, r, pl.ds(nb * 128 + g * 16, 16)] = vals
          return carry3

        lax.fori_loop(0, 8, g_body, 0)
        return carry2

      lax.fori_loop(0, 32, cr_body, 0)
      return carry

    lax.fori_loop(0, NB, nb_body, 0)

    def pw_body(p, carry):
      for cb in range(4):
        pltpu.async_copy(roadblk.at[cb],
                         out_hbm.at[p, pl.ds(cb * 8, 8), pl.ds(n0, NB * 128)],
                         wsem)
      for cb in range(4):
        pltpu.make_async_copy(
            roadblk.at[cb],
            out_hbm.at[p, pl.ds(cb * 8, 8), pl.ds(n0, NB * 128)],
            wsem).wait()
      return carry

    lax.fori_loop(0, P, pw_body, 0)

  return k(road_t, dt_t, xr, xd)


def kernel(x_road, x_datetime, road_table, datetime_table):
  road_t = road_table.T
  dt_t = datetime_table.T
  xr = x_road.reshape(N).astype(jnp.int32)
  xd = jnp.swapaxes(x_datetime, 0, 1).reshape(P * N).astype(jnp.int32)
  out, _ = _sc_embed(road_t, dt_t, xr, xd)
  return jnp.transpose(out, (2, 0, 1))
